# baseline (speedup 1.0000x reference)
import functools
import jax
import jax.numpy as jnp
from jax import lax
from jax.experimental import pallas as pl
from jax.experimental.pallas import tpu as pltpu
from jax.experimental.pallas import tpu_sc as plsc
from jax._src.pallas import mpmd as _mpmd

EMB = 1000
LANES = 16
NC = 2
NS = 16
NW = NC * NS


ZCHUNK = 8000


def _zero_body(out_ref, buf, sem):
    nrows = out_ref.shape[0]
    nchunks = nrows // ZCHUNK
    buf[:, :] = jnp.zeros((ZCHUNK, 128), jnp.float32)
    for i in range(nchunks):
        pltpu.make_async_copy(
            buf, out_ref.at[pl.ds(i * ZCHUNK, ZCHUNK), :], sem
        ).start()
    for i in range(nchunks):
        pltpu.make_async_copy(
            buf, out_ref.at[pl.ds(i * ZCHUNK, ZCHUNK), :], sem
        ).wait()


def _zero_fill(batch):
    return pl.pallas_call(
        _zero_body,
        out_specs=pl.BlockSpec(memory_space=pl.ANY),
        out_shape=jax.ShapeDtypeStruct((batch * EMB // 128, 128), jnp.float32),
        scratch_shapes=[
            pltpu.VMEM((ZCHUNK, 128), jnp.float32),
            pltpu.SemaphoreType.DMA,
        ],
    )()


def _scatter_body(labels_hbm, flat_in, flat_out, labels_v, pos_v, ones_v, sem):
    del flat_in
    rows_per_w = labels_hbm.shape[0] // NW
    wid = lax.axis_index("s") * NC + lax.axis_index("c")
    base = wid * rows_per_w
    pltpu.sync_copy(labels_hbm.at[pl.ds(base, rows_per_w)], labels_v)
    lane = lax.iota(jnp.int32, LANES)
    for j in range(rows_per_w // LANES):
        r = base + j * LANES + lane
        labs = labels_v[pl.ds(j * LANES, LANES)]
        pos_v[pl.ds(j * LANES, LANES)] = r * EMB + labs
        ones_v[pl.ds(j * LANES, LANES)] = jnp.full((LANES,), 1.0, jnp.float32)
    pltpu.async_copy(ones_v, flat_out.at[pos_v], sem).wait()


def _sc_scatter(labels, flat):
    rows_per_w = labels.shape[0] // NW
    mesh = plsc.VectorSubcoreMesh(core_axis_name="c", subcore_axis_name="s")
    return _mpmd._mpmd_map(
        [(mesh, _scatter_body)],
        jax.ShapeDtypeStruct(flat.shape, flat.dtype),
        input_output_aliases={1: 0},
        scratch_types=[
            pltpu.VMEM((rows_per_w,), jnp.int32),
            pltpu.VMEM((rows_per_w,), jnp.int32),
            pltpu.VMEM((rows_per_w,), jnp.float32),
            pltpu.SemaphoreType.DMA,
        ],
    )(labels, flat)


def kernel(labels):
    batch = labels.shape[0]
    flat = _zero_fill(batch).reshape(batch * EMB)
    out = _sc_scatter(labels.astype(jnp.int32), flat)
    return out.reshape(batch, EMB)


# speedup vs baseline: 8.2701x; 8.2701x over previous
import jax
import jax.numpy as jnp
from jax.experimental import pallas as pl

EMB = 1000
CB = 1024


def _onehot_t_block(labels_ref, out_ref):
    labs = labels_ref[:].astype(jnp.int32)
    rows = jax.lax.broadcasted_iota(jnp.int32, (EMB, CB), 0)
    out_ref[:, :] = (rows == labs[None, :]).astype(jnp.float32)


def kernel(labels):
    batch = labels.shape[0]
    grid = batch // CB
    out_t = pl.pallas_call(
        _onehot_t_block,
        grid=(grid,),
        in_specs=[pl.BlockSpec((CB,), lambda i: (i,))],
        out_specs=pl.BlockSpec((EMB, CB), lambda i: (0, i)),
        out_shape=jax.ShapeDtypeStruct((EMB, batch), jnp.float32),
    )(labels)
    return out_t.T
